# baseline (speedup 1.0000x reference)
import functools

import jax
import jax.numpy as jnp
from jax import lax
from jax.experimental import pallas as pl
from jax.experimental.pallas import tpu as pltpu
from jax.experimental.pallas import tpu_sc as plsc

NUM_NODES = 1000000
DIM = 32
BATCH = 16384

_INFO = plsc.get_sparse_core_info()
_NC = _INFO.num_cores
_NS = _INFO.num_subcores
_NW = _NC * _NS
_LANES = _INFO.num_lanes

_BPW = BATCH // _NW
_CHUNK = 128
_NCHUNK = _BPW // _CHUNK
_GROUPS = _BPW // _LANES


def _sc_kernel(emb_hbm, i_hbm, j_hbm, out_hbm,
               idx_i_v, idx_j_v, vi_v, vj_v, out_v, sem):
    wid = lax.axis_index("s") * _NC + lax.axis_index("c")
    row0 = wid * _NCHUNK

    pltpu.sync_copy(i_hbm.at[pl.ds(row0, _NCHUNK)], idx_i_v)
    pltpu.sync_copy(j_hbm.at[pl.ds(row0, _NCHUNK)], idx_j_v)

    handles = []
    for k in range(_NCHUNK):
        handles.append(pltpu.async_copy(
            emb_hbm.at[idx_i_v.at[k]],
            vi_v.at[pl.ds(k * _CHUNK, _CHUNK)], sem))
        handles.append(pltpu.async_copy(
            emb_hbm.at[idx_j_v.at[k]],
            vj_v.at[pl.ds(k * _CHUNK, _CHUNK)], sem))
    for h in handles:
        h.wait()

    def body(g, _):
        rows = g * _LANES + lax.iota(jnp.int32, _LANES)
        acc = jnp.zeros((_LANES,), jnp.float32)
        for d in range(DIM):
            col = jnp.full((_LANES,), d, jnp.int32)
            a = plsc.load_gather(vi_v, [rows, col])
            b = plsc.load_gather(vj_v, [rows, col])
            acc = acc + a * b
        out_v[pl.ds(g * _LANES, _LANES)] = acc
        return 0

    lax.fori_loop(0, _GROUPS, body, 0)

    pltpu.sync_copy(out_v, out_hbm.at[pl.ds(wid * _BPW, _BPW)])


@jax.jit
def kernel(i, j, node_emb):
    i2 = i.astype(jnp.int32).reshape(_NW * _NCHUNK, _CHUNK)
    j2 = j.astype(jnp.int32).reshape(_NW * _NCHUNK, _CHUNK)
    mesh = plsc.VectorSubcoreMesh(core_axis_name="c", subcore_axis_name="s")
    fn = functools.partial(
        pl.kernel,
        mesh=mesh,
        out_type=jax.ShapeDtypeStruct((BATCH,), jnp.float32),
        scratch_types=[
            pltpu.VMEM((_NCHUNK, _CHUNK), jnp.int32),
            pltpu.VMEM((_NCHUNK, _CHUNK), jnp.int32),
            pltpu.VMEM((_BPW, DIM), jnp.float32),
            pltpu.VMEM((_BPW, DIM), jnp.float32),
            pltpu.VMEM((_BPW,), jnp.float32),
            pltpu.SemaphoreType.DMA,
        ],
        compiler_params=pltpu.CompilerParams(
            use_tc_tiling_on_sc=False, needs_layout_passes=False),
    )(_sc_kernel)
    return fn(node_emb, i2, j2)


# speedup vs baseline: 1.7232x; 1.7232x over previous
import functools

import jax
import jax.numpy as jnp
from jax import lax
from jax.experimental import pallas as pl
from jax.experimental.pallas import tpu as pltpu
from jax.experimental.pallas import tpu_sc as plsc

NUM_NODES = 1000000
DIM = 32
BATCH = 16384

_Q = 1 << 18
_PACK = 4
_NROWS = _Q

_INFO = plsc.get_sparse_core_info()
_NC = _INFO.num_cores
_NS = _INFO.num_subcores
_NW = _NC * _NS
_LANES = _INFO.num_lanes

_BPW = BATCH // _NW
_CHUNK = 128
_NCHUNK = _BPW // _CHUNK
_GRP = _CHUNK // _LANES

_TC_R = 2048
_TC_G = _NROWS // _TC_R


def _tc_relayout_body(x0, x1, x2, x3, o_ref):
    o_ref[...] = jnp.concatenate(
        [x0[...].T, x1[...].T, x2[...].T, x3[...].T], axis=1)


def _tc_relayout(embT):
    last_ok = -(-NUM_NODES // _TC_R) - 1
    specs = [
        pl.BlockSpec((DIM, _TC_R), functools.partial(
            lambda m, g: (0, jnp.minimum(m * _TC_G + g, last_ok)), m))
        for m in range(_PACK)
    ]
    return pl.pallas_call(
        _tc_relayout_body,
        grid=(_TC_G,),
        in_specs=specs,
        out_specs=pl.BlockSpec((_TC_R, _PACK * DIM), lambda g: (g, 0)),
        out_shape=jax.ShapeDtypeStruct((_NROWS, _PACK * DIM), jnp.float32),
    )(embT, embT, embT, embT)


def _sc_kernel(emb_hbm, i_hbm, j_hbm, out_hbm,
               idx_i_v, idx_j_v, qi_v, qj_v, ci_v, cj_v,
               a0_v, b0_v, a1_v, b1_v, out_v, sem0, sem1):
    wid = lax.axis_index("s") * _NC + lax.axis_index("c")
    row0 = wid * _NCHUNK

    pltpu.sync_copy(i_hbm.at[pl.ds(row0, _NCHUNK)], idx_i_v)
    pltpu.sync_copy(j_hbm.at[pl.ds(row0, _NCHUNK)], idx_j_v)

    for k in range(_NCHUNK):
        for c in range(_GRP):
            s = pl.ds(c * _LANES, _LANES)
            vi = idx_i_v[k, s]
            qi_v[k, s] = vi & (_Q - 1)
            ci_v[k, s] = (vi >> 18) << 5
            vj = idx_j_v[k, s]
            qj_v[k, s] = vj & (_Q - 1)
            cj_v[k, s] = (vj >> 18) << 5

    bufs = ((a0_v, b0_v, sem0), (a1_v, b1_v, sem1))

    def fire(k):
        a_v, b_v, sem = bufs[k % 2]
        return (pltpu.async_copy(emb_hbm.at[qi_v.at[k]], a_v, sem),
                pltpu.async_copy(emb_hbm.at[qj_v.at[k]], b_v, sem))

    pending = fire(0)
    for k in range(_NCHUNK):
        nxt = fire(k + 1) if k + 1 < _NCHUNK else None
        for h in pending:
            h.wait()
        a_v, b_v, _ = bufs[k % 2]

        def body(g, _, k=k, a_v=a_v, b_v=b_v):
            s = pl.ds(g * _LANES, _LANES)
            rows = g * _LANES + lax.iota(jnp.int32, _LANES)
            cbi = ci_v[k, s]
            cbj = cj_v[k, s]
            acc = jnp.zeros((_LANES,), jnp.float32)
            for d in range(DIM):
                a = plsc.load_gather(a_v, [rows, cbi + d])
                b = plsc.load_gather(b_v, [rows, cbj + d])
                acc = acc + a * b
            out_v[k, s] = acc
            return 0

        lax.fori_loop(0, _GRP, body, 0)
        pending = nxt

    pltpu.sync_copy(out_v, out_hbm.at[pl.ds(row0, _NCHUNK)])


@jax.jit
def kernel(i, j, node_emb):
    embT = node_emb.T
    packed = _tc_relayout(embT)
    i2 = i.astype(jnp.int32).reshape(_NW * _NCHUNK, _CHUNK)
    j2 = j.astype(jnp.int32).reshape(_NW * _NCHUNK, _CHUNK)
    mesh = plsc.VectorSubcoreMesh(core_axis_name="c", subcore_axis_name="s")
    fn = functools.partial(
        pl.kernel,
        mesh=mesh,
        out_type=jax.ShapeDtypeStruct((_NW * _NCHUNK, _CHUNK), jnp.float32),
        scratch_types=[
            pltpu.VMEM((_NCHUNK, _CHUNK), jnp.int32),
            pltpu.VMEM((_NCHUNK, _CHUNK), jnp.int32),
            pltpu.VMEM((_NCHUNK, _CHUNK), jnp.int32),
            pltpu.VMEM((_NCHUNK, _CHUNK), jnp.int32),
            pltpu.VMEM((_NCHUNK, _CHUNK), jnp.int32),
            pltpu.VMEM((_NCHUNK, _CHUNK), jnp.int32),
            pltpu.VMEM((_CHUNK, _PACK * DIM), jnp.float32),
            pltpu.VMEM((_CHUNK, _PACK * DIM), jnp.float32),
            pltpu.VMEM((_CHUNK, _PACK * DIM), jnp.float32),
            pltpu.VMEM((_CHUNK, _PACK * DIM), jnp.float32),
            pltpu.VMEM((_NCHUNK, _CHUNK), jnp.float32),
            pltpu.SemaphoreType.DMA,
            pltpu.SemaphoreType.DMA,
        ],
        compiler_params=pltpu.CompilerParams(
            use_tc_tiling_on_sc=False, needs_layout_passes=False),
    )(_sc_kernel)
    out2 = fn(packed, i2, j2)
    return out2.reshape(BATCH)


# speedup vs baseline: 2.9407x; 1.7065x over previous
import functools

import jax
import jax.numpy as jnp
from jax import lax
from jax.experimental import pallas as pl
from jax.experimental.pallas import tpu as pltpu
from jax.experimental.pallas import tpu_sc as plsc

NUM_NODES = 1000000
DIM = 32
BATCH = 16384

_Q = 1 << 18
_PACK = 4
_NROWS = _Q

_INFO = plsc.get_sparse_core_info()
_NC = _INFO.num_cores
_NS = _INFO.num_subcores
_NW = _NC * _NS
_LANES = _INFO.num_lanes

_BPW = BATCH // _NW
_CHUNK = 128
_NCHUNK = _BPW // _CHUNK
_GRP = _CHUNK // _LANES

_TC_R = 2048
_TC_G = _NROWS // _TC_R


def _tc_relayout_body(x0, x1, x2, x3, o_ref):
    o_ref[...] = jnp.concatenate(
        [x0[...], x1[...], x2[...], x3[...]], axis=0).T


def _tc_relayout(embT):
    last_ok = -(-NUM_NODES // _TC_R) - 1
    specs = [
        pl.BlockSpec((DIM, _TC_R), functools.partial(
            lambda m, g: (0, jnp.minimum(m * _TC_G + g, last_ok)), m))
        for m in range(_PACK)
    ]
    return pl.pallas_call(
        _tc_relayout_body,
        grid=(_TC_G,),
        in_specs=specs,
        out_specs=pl.BlockSpec((_TC_R, _PACK * DIM), lambda g: (g, 0)),
        out_shape=jax.ShapeDtypeStruct((_NROWS, _PACK * DIM), jnp.float32),
    )(embT, embT, embT, embT)


def _sc_kernel(emb_hbm, i_hbm, j_hbm, out_hbm,
               idx_i_v, idx_j_v, qi_v, qj_v, ci_v, cj_v,
               a0_v, b0_v, a1_v, b1_v, out_v, sem0, sem1):
    wid = lax.axis_index("s") * _NC + lax.axis_index("c")
    row0 = wid * _NCHUNK

    pltpu.sync_copy(i_hbm.at[pl.ds(row0, _NCHUNK)], idx_i_v)
    pltpu.sync_copy(j_hbm.at[pl.ds(row0, _NCHUNK)], idx_j_v)

    for k in range(_NCHUNK):
        for c in range(_GRP):
            s = pl.ds(c * _LANES, _LANES)
            vi = idx_i_v[k, s]
            qi_v[k, s] = vi & (_Q - 1)
            ci_v[k, s] = (vi >> 18) << 5
            vj = idx_j_v[k, s]
            qj_v[k, s] = vj & (_Q - 1)
            cj_v[k, s] = (vj >> 18) << 5

    bufs = ((a0_v, b0_v, sem0), (a1_v, b1_v, sem1))

    def fire(k):
        a_v, b_v, sem = bufs[k % 2]
        return (pltpu.async_copy(emb_hbm.at[qi_v.at[k]], a_v, sem),
                pltpu.async_copy(emb_hbm.at[qj_v.at[k]], b_v, sem))

    pending = fire(0)
    for k in range(_NCHUNK):
        nxt = fire(k + 1) if k + 1 < _NCHUNK else None
        for h in pending:
            h.wait()
        a_v, b_v, _ = bufs[k % 2]

        def body(g, _, k=k, a_v=a_v, b_v=b_v):
            s = pl.ds(g * _LANES, _LANES)
            rows = g * _LANES + lax.iota(jnp.int32, _LANES)
            cbi = ci_v[k, s]
            cbj = cj_v[k, s]
            acc = jnp.zeros((_LANES,), jnp.float32)
            for d in range(DIM):
                a = plsc.load_gather(a_v, [rows, cbi + d])
                b = plsc.load_gather(b_v, [rows, cbj + d])
                acc = acc + a * b
            out_v[k, s] = acc
            return 0

        lax.fori_loop(0, _GRP, body, 0)
        pending = nxt

    pltpu.sync_copy(out_v, out_hbm.at[pl.ds(row0, _NCHUNK)])


@jax.jit
def kernel(i, j, node_emb):
    embT = node_emb.T
    packed = _tc_relayout(embT)
    i2 = i.astype(jnp.int32).reshape(_NW * _NCHUNK, _CHUNK)
    j2 = j.astype(jnp.int32).reshape(_NW * _NCHUNK, _CHUNK)
    mesh = plsc.VectorSubcoreMesh(core_axis_name="c", subcore_axis_name="s")
    fn = functools.partial(
        pl.kernel,
        mesh=mesh,
        out_type=jax.ShapeDtypeStruct((_NW * _NCHUNK, _CHUNK), jnp.float32),
        scratch_types=[
            pltpu.VMEM((_NCHUNK, _CHUNK), jnp.int32),
            pltpu.VMEM((_NCHUNK, _CHUNK), jnp.int32),
            pltpu.VMEM((_NCHUNK, _CHUNK), jnp.int32),
            pltpu.VMEM((_NCHUNK, _CHUNK), jnp.int32),
            pltpu.VMEM((_NCHUNK, _CHUNK), jnp.int32),
            pltpu.VMEM((_NCHUNK, _CHUNK), jnp.int32),
            pltpu.VMEM((_CHUNK, _PACK * DIM), jnp.float32),
            pltpu.VMEM((_CHUNK, _PACK * DIM), jnp.float32),
            pltpu.VMEM((_CHUNK, _PACK * DIM), jnp.float32),
            pltpu.VMEM((_CHUNK, _PACK * DIM), jnp.float32),
            pltpu.VMEM((_NCHUNK, _CHUNK), jnp.float32),
            pltpu.SemaphoreType.DMA,
            pltpu.SemaphoreType.DMA,
        ],
        compiler_params=pltpu.CompilerParams(
            use_tc_tiling_on_sc=False, needs_layout_passes=False),
    )(_sc_kernel)
    out2 = fn(packed, i2, j2)
    return out2.reshape(BATCH)


# speedup vs baseline: 3.7400x; 1.2718x over previous
import functools

import jax
import jax.numpy as jnp
from jax import lax
from jax.experimental import pallas as pl
from jax.experimental.pallas import tpu as pltpu
from jax.experimental.pallas import tpu_sc as plsc

NUM_NODES = 1000000
DIM = 32
BATCH = 16384

_Q = 1 << 18
_PACK = 4
_NROWS = _Q

_INFO = plsc.get_sparse_core_info()
_NC = _INFO.num_cores
_NS = _INFO.num_subcores
_NW = _NC * _NS
_LANES = _INFO.num_lanes

_BPW = BATCH // _NW
_CHUNK = 128
_NCHUNK = _BPW // _CHUNK
_GRP = _CHUNK // _LANES

_TC_R = 4096
_TC_G = _NROWS // _TC_R


def _tc_relayout_body(x0, x1, x2, x3, o_ref):
    o_ref[...] = jnp.concatenate(
        [x0[...], x1[...], x2[...], x3[...]], axis=0).T


def _tc_relayout(embT):
    last_ok = -(-NUM_NODES // _TC_R) - 1
    specs = [
        pl.BlockSpec((DIM, _TC_R), functools.partial(
            lambda m, g: (0, jnp.minimum(m * _TC_G + g, last_ok)), m))
        for m in range(_PACK)
    ]
    return pl.pallas_call(
        _tc_relayout_body,
        grid=(_TC_G,),
        in_specs=specs,
        out_specs=pl.BlockSpec((_TC_R, _PACK * DIM), lambda g: (g, 0)),
        out_shape=jax.ShapeDtypeStruct((_NROWS, _PACK * DIM), jnp.float32),
    )(embT, embT, embT, embT)


def _sc_kernel(emb_hbm, i_hbm, j_hbm, out_hbm,
               idx_i_v, idx_j_v, qi_v, qj_v, ci_v, cj_v,
               a0_v, b0_v, a1_v, b1_v, out_v, sem0, sem1):
    wid = lax.axis_index("s") * _NC + lax.axis_index("c")
    row0 = wid * _NCHUNK

    pltpu.sync_copy(i_hbm.at[pl.ds(row0, _NCHUNK)], idx_i_v)
    pltpu.sync_copy(j_hbm.at[pl.ds(row0, _NCHUNK)], idx_j_v)

    for k in range(_NCHUNK):
        for c in range(_GRP):
            s = pl.ds(c * _LANES, _LANES)
            vi = idx_i_v[k, s]
            qi_v[k, s] = vi & (_Q - 1)
            ci_v[k, s] = (vi >> 18) << 5
            vj = idx_j_v[k, s]
            qj_v[k, s] = vj & (_Q - 1)
            cj_v[k, s] = (vj >> 18) << 5

    bufs = ((a0_v, b0_v, sem0), (a1_v, b1_v, sem1))

    def fire(k):
        a_v, b_v, sem = bufs[k % 2]
        return (pltpu.async_copy(emb_hbm.at[qi_v.at[k]], a_v, sem),
                pltpu.async_copy(emb_hbm.at[qj_v.at[k]], b_v, sem))

    pending = fire(0)
    for k in range(_NCHUNK):
        nxt = fire(k + 1) if k + 1 < _NCHUNK else None
        for h in pending:
            h.wait()
        a_v, b_v, _ = bufs[k % 2]

        def body(g, _, k=k, a_v=a_v, b_v=b_v):
            s = pl.ds(g * _LANES, _LANES)
            rows = g * _LANES + lax.iota(jnp.int32, _LANES)
            cbi = ci_v[k, s]
            cbj = cj_v[k, s]
            acc = jnp.zeros((_LANES,), jnp.float32)
            for d in range(DIM):
                a = plsc.load_gather(a_v, [rows, cbi + d])
                b = plsc.load_gather(b_v, [rows, cbj + d])
                acc = acc + a * b
            out_v[k, s] = acc
            return 0

        lax.fori_loop(0, _GRP, body, 0)
        pending = nxt

    pltpu.sync_copy(out_v, out_hbm.at[pl.ds(row0, _NCHUNK)])


@jax.jit
def kernel(i, j, node_emb):
    embT = node_emb.T
    packed = _tc_relayout(embT)
    i2 = i.astype(jnp.int32).reshape(_NW * _NCHUNK, _CHUNK)
    j2 = j.astype(jnp.int32).reshape(_NW * _NCHUNK, _CHUNK)
    mesh = plsc.VectorSubcoreMesh(core_axis_name="c", subcore_axis_name="s")
    fn = functools.partial(
        pl.kernel,
        mesh=mesh,
        out_type=jax.ShapeDtypeStruct((_NW * _NCHUNK, _CHUNK), jnp.float32),
        scratch_types=[
            pltpu.VMEM((_NCHUNK, _CHUNK), jnp.int32),
            pltpu.VMEM((_NCHUNK, _CHUNK), jnp.int32),
            pltpu.VMEM((_NCHUNK, _CHUNK), jnp.int32),
            pltpu.VMEM((_NCHUNK, _CHUNK), jnp.int32),
            pltpu.VMEM((_NCHUNK, _CHUNK), jnp.int32),
            pltpu.VMEM((_NCHUNK, _CHUNK), jnp.int32),
            pltpu.VMEM((_CHUNK, _PACK * DIM), jnp.float32),
            pltpu.VMEM((_CHUNK, _PACK * DIM), jnp.float32),
            pltpu.VMEM((_CHUNK, _PACK * DIM), jnp.float32),
            pltpu.VMEM((_CHUNK, _PACK * DIM), jnp.float32),
            pltpu.VMEM((_NCHUNK, _CHUNK), jnp.float32),
            pltpu.SemaphoreType.DMA,
            pltpu.SemaphoreType.DMA,
        ],
        compiler_params=pltpu.CompilerParams(
            use_tc_tiling_on_sc=False, needs_layout_passes=False),
    )(_sc_kernel)
    out2 = fn(packed, i2, j2)
    return out2.reshape(BATCH)


# speedup vs baseline: 4.1571x; 1.1115x over previous
import functools

import jax
import jax.numpy as jnp
from jax import lax
from jax.experimental import pallas as pl
from jax.experimental.pallas import tpu as pltpu
from jax.experimental.pallas import tpu_sc as plsc

NUM_NODES = 1000000
DIM = 32
BATCH = 16384

_Q = 1 << 18
_PACK = 4
_NROWS = _Q

_INFO = plsc.get_sparse_core_info()
_NC = _INFO.num_cores
_NS = _INFO.num_subcores
_NW = _NC * _NS
_LANES = _INFO.num_lanes

_BPW = BATCH // _NW
_CHUNK = 128
_NCHUNK = _BPW // _CHUNK
_GRP = _CHUNK // _LANES

_TC_R = 8192
_TC_G = _NROWS // _TC_R


def _tc_relayout_body(x0, x1, x2, x3, o_ref):
    o_ref[...] = jnp.concatenate(
        [x0[...], x1[...], x2[...], x3[...]], axis=0).T


def _tc_relayout(embT):
    last_ok = -(-NUM_NODES // _TC_R) - 1
    specs = [
        pl.BlockSpec((DIM, _TC_R), functools.partial(
            lambda m, g: (0, jnp.minimum(m * _TC_G + g, last_ok)), m))
        for m in range(_PACK)
    ]
    return pl.pallas_call(
        _tc_relayout_body,
        grid=(_TC_G,),
        in_specs=specs,
        out_specs=pl.BlockSpec((_TC_R, _PACK * DIM), lambda g: (g, 0)),
        out_shape=jax.ShapeDtypeStruct((_NROWS, _PACK * DIM), jnp.float32),
    )(embT, embT, embT, embT)


def _sc_kernel(emb_hbm, i_hbm, j_hbm, out_hbm,
               idx_i_v, idx_j_v, qi_v, qj_v, ci_v, cj_v,
               a0_v, b0_v, a1_v, b1_v, out_v, sem0, sem1):
    wid = lax.axis_index("s") * _NC + lax.axis_index("c")
    row0 = wid * _NCHUNK

    pltpu.sync_copy(i_hbm.at[pl.ds(row0, _NCHUNK)], idx_i_v)
    pltpu.sync_copy(j_hbm.at[pl.ds(row0, _NCHUNK)], idx_j_v)

    for k in range(_NCHUNK):
        for c in range(_GRP):
            s = pl.ds(c * _LANES, _LANES)
            vi = idx_i_v[k, s]
            qi_v[k, s] = vi & (_Q - 1)
            ci_v[k, s] = (vi >> 18) << 5
            vj = idx_j_v[k, s]
            qj_v[k, s] = vj & (_Q - 1)
            cj_v[k, s] = (vj >> 18) << 5

    bufs = ((a0_v, b0_v, sem0), (a1_v, b1_v, sem1))

    def fire(k):
        a_v, b_v, sem = bufs[k % 2]
        return (pltpu.async_copy(emb_hbm.at[qi_v.at[k]], a_v, sem),
                pltpu.async_copy(emb_hbm.at[qj_v.at[k]], b_v, sem))

    pending = fire(0)
    for k in range(_NCHUNK):
        nxt = fire(k + 1) if k + 1 < _NCHUNK else None
        for h in pending:
            h.wait()
        a_v, b_v, _ = bufs[k % 2]

        def body(g, _, k=k, a_v=a_v, b_v=b_v):
            s = pl.ds(g * _LANES, _LANES)
            rows = g * _LANES + lax.iota(jnp.int32, _LANES)
            cbi = ci_v[k, s]
            cbj = cj_v[k, s]
            acc = jnp.zeros((_LANES,), jnp.float32)
            for d in range(DIM):
                a = plsc.load_gather(a_v, [rows, cbi + d])
                b = plsc.load_gather(b_v, [rows, cbj + d])
                acc = acc + a * b
            out_v[k, s] = acc
            return 0

        lax.fori_loop(0, _GRP, body, 0)
        pending = nxt

    pltpu.sync_copy(out_v, out_hbm.at[pl.ds(row0, _NCHUNK)])


@jax.jit
def kernel(i, j, node_emb):
    embT = node_emb.T
    packed = _tc_relayout(embT)
    i2 = i.astype(jnp.int32).reshape(_NW * _NCHUNK, _CHUNK)
    j2 = j.astype(jnp.int32).reshape(_NW * _NCHUNK, _CHUNK)
    mesh = plsc.VectorSubcoreMesh(core_axis_name="c", subcore_axis_name="s")
    fn = functools.partial(
        pl.kernel,
        mesh=mesh,
        out_type=jax.ShapeDtypeStruct((_NW * _NCHUNK, _CHUNK), jnp.float32),
        scratch_types=[
            pltpu.VMEM((_NCHUNK, _CHUNK), jnp.int32),
            pltpu.VMEM((_NCHUNK, _CHUNK), jnp.int32),
            pltpu.VMEM((_NCHUNK, _CHUNK), jnp.int32),
            pltpu.VMEM((_NCHUNK, _CHUNK), jnp.int32),
            pltpu.VMEM((_NCHUNK, _CHUNK), jnp.int32),
            pltpu.VMEM((_NCHUNK, _CHUNK), jnp.int32),
            pltpu.VMEM((_CHUNK, _PACK * DIM), jnp.float32),
            pltpu.VMEM((_CHUNK, _PACK * DIM), jnp.float32),
            pltpu.VMEM((_CHUNK, _PACK * DIM), jnp.float32),
            pltpu.VMEM((_CHUNK, _PACK * DIM), jnp.float32),
            pltpu.VMEM((_NCHUNK, _CHUNK), jnp.float32),
            pltpu.SemaphoreType.DMA,
            pltpu.SemaphoreType.DMA,
        ],
        compiler_params=pltpu.CompilerParams(
            use_tc_tiling_on_sc=False, needs_layout_passes=False),
    )(_sc_kernel)
    out2 = fn(packed, i2, j2)
    return out2.reshape(BATCH)


# speedup vs baseline: 4.2543x; 1.0234x over previous
import functools

import jax
import jax.numpy as jnp
from jax import lax
from jax.experimental import pallas as pl
from jax.experimental.pallas import tpu as pltpu
from jax.experimental.pallas import tpu_sc as plsc

NUM_NODES = 1000000
DIM = 32
BATCH = 16384

_Q = 1 << 18
_PACK = 4
_NROWS = _Q

_INFO = plsc.get_sparse_core_info()
_NC = _INFO.num_cores
_NS = _INFO.num_subcores
_NW = _NC * _NS
_LANES = _INFO.num_lanes

_BPW = BATCH // _NW
_CHUNK = 128
_NCHUNK = _BPW // _CHUNK
_GRP = _CHUNK // _LANES

_TC_R = 16384
_TC_G = _NROWS // _TC_R


def _tc_relayout_body(x0, x1, x2, x3, o_ref):
    o_ref[...] = jnp.concatenate(
        [x0[...], x1[...], x2[...], x3[...]], axis=0).T


def _tc_relayout(embT):
    last_ok = -(-NUM_NODES // _TC_R) - 1
    specs = [
        pl.BlockSpec((DIM, _TC_R), functools.partial(
            lambda m, g: (0, jnp.minimum(m * _TC_G + g, last_ok)), m))
        for m in range(_PACK)
    ]
    return pl.pallas_call(
        _tc_relayout_body,
        grid=(_TC_G,),
        in_specs=specs,
        out_specs=pl.BlockSpec((_TC_R, _PACK * DIM), lambda g: (g, 0)),
        out_shape=jax.ShapeDtypeStruct((_NROWS, _PACK * DIM), jnp.float32),
    )(embT, embT, embT, embT)


def _sc_kernel(emb_hbm, i_hbm, j_hbm, out_hbm,
               idx_i_v, idx_j_v, qi_v, qj_v, ci_v, cj_v,
               a0_v, b0_v, a1_v, b1_v, out_v, sem0, sem1):
    wid = lax.axis_index("s") * _NC + lax.axis_index("c")
    row0 = wid * _NCHUNK

    pltpu.sync_copy(i_hbm.at[pl.ds(row0, _NCHUNK)], idx_i_v)
    pltpu.sync_copy(j_hbm.at[pl.ds(row0, _NCHUNK)], idx_j_v)

    for k in range(_NCHUNK):
        for c in range(_GRP):
            s = pl.ds(c * _LANES, _LANES)
            vi = idx_i_v[k, s]
            qi_v[k, s] = vi & (_Q - 1)
            ci_v[k, s] = (vi >> 18) << 5
            vj = idx_j_v[k, s]
            qj_v[k, s] = vj & (_Q - 1)
            cj_v[k, s] = (vj >> 18) << 5

    bufs = ((a0_v, b0_v, sem0), (a1_v, b1_v, sem1))

    def fire(k):
        a_v, b_v, sem = bufs[k % 2]
        return (pltpu.async_copy(emb_hbm.at[qi_v.at[k]], a_v, sem),
                pltpu.async_copy(emb_hbm.at[qj_v.at[k]], b_v, sem))

    pending = fire(0)
    for k in range(_NCHUNK):
        nxt = fire(k + 1) if k + 1 < _NCHUNK else None
        for h in pending:
            h.wait()
        a_v, b_v, _ = bufs[k % 2]

        def body(g, _, k=k, a_v=a_v, b_v=b_v):
            s = pl.ds(g * _LANES, _LANES)
            rows = g * _LANES + lax.iota(jnp.int32, _LANES)
            cbi = ci_v[k, s]
            cbj = cj_v[k, s]
            acc = jnp.zeros((_LANES,), jnp.float32)
            for d in range(DIM):
                a = plsc.load_gather(a_v, [rows, cbi + d])
                b = plsc.load_gather(b_v, [rows, cbj + d])
                acc = acc + a * b
            out_v[k, s] = acc
            return 0

        lax.fori_loop(0, _GRP, body, 0)
        pending = nxt

    pltpu.sync_copy(out_v, out_hbm.at[pl.ds(row0, _NCHUNK)])


@jax.jit
def kernel(i, j, node_emb):
    embT = node_emb.T
    packed = _tc_relayout(embT)
    i2 = i.astype(jnp.int32).reshape(_NW * _NCHUNK, _CHUNK)
    j2 = j.astype(jnp.int32).reshape(_NW * _NCHUNK, _CHUNK)
    mesh = plsc.VectorSubcoreMesh(core_axis_name="c", subcore_axis_name="s")
    fn = functools.partial(
        pl.kernel,
        mesh=mesh,
        out_type=jax.ShapeDtypeStruct((_NW * _NCHUNK, _CHUNK), jnp.float32),
        scratch_types=[
            pltpu.VMEM((_NCHUNK, _CHUNK), jnp.int32),
            pltpu.VMEM((_NCHUNK, _CHUNK), jnp.int32),
            pltpu.VMEM((_NCHUNK, _CHUNK), jnp.int32),
            pltpu.VMEM((_NCHUNK, _CHUNK), jnp.int32),
            pltpu.VMEM((_NCHUNK, _CHUNK), jnp.int32),
            pltpu.VMEM((_NCHUNK, _CHUNK), jnp.int32),
            pltpu.VMEM((_CHUNK, _PACK * DIM), jnp.float32),
            pltpu.VMEM((_CHUNK, _PACK * DIM), jnp.float32),
            pltpu.VMEM((_CHUNK, _PACK * DIM), jnp.float32),
            pltpu.VMEM((_CHUNK, _PACK * DIM), jnp.float32),
            pltpu.VMEM((_NCHUNK, _CHUNK), jnp.float32),
            pltpu.SemaphoreType.DMA,
            pltpu.SemaphoreType.DMA,
        ],
        compiler_params=pltpu.CompilerParams(
            use_tc_tiling_on_sc=False, needs_layout_passes=False),
    )(_sc_kernel)
    out2 = fn(packed, i2, j2)
    return out2.reshape(BATCH)


# speedup vs baseline: 4.3767x; 1.0288x over previous
import functools

import jax
import jax.numpy as jnp
from jax import lax
from jax.experimental import pallas as pl
from jax.experimental.pallas import tpu as pltpu
from jax.experimental.pallas import tpu_sc as plsc

NUM_NODES = 1000000
DIM = 32
BATCH = 16384

_Q = 1 << 18
_PACK = 4
_NROWS = _Q

_INFO = plsc.get_sparse_core_info()
_NC = _INFO.num_cores
_NS = _INFO.num_subcores
_NW = _NC * _NS
_LANES = _INFO.num_lanes

_BPW = BATCH // _NW
_CHUNK = 128
_NCHUNK = _BPW // _CHUNK
_GRP = _CHUNK // _LANES

_TC_R = 16384
_TC_G = _NROWS // _TC_R


def _tc_relayout_body(x0, x1, x2, x3, o_ref):
    o_ref[...] = jnp.concatenate(
        [x0[...], x1[...], x2[...], x3[...]], axis=0).T


def _tc_relayout(embT):
    last_ok = -(-NUM_NODES // _TC_R) - 1
    specs = [
        pl.BlockSpec((DIM, _TC_R), functools.partial(
            lambda m, g: (0, jnp.minimum(m * _TC_G + g, last_ok)), m))
        for m in range(_PACK)
    ]
    return pl.pallas_call(
        _tc_relayout_body,
        grid=(_TC_G,),
        in_specs=specs,
        out_specs=pl.BlockSpec((_TC_R, _PACK * DIM), lambda g: (g, 0)),
        out_shape=jax.ShapeDtypeStruct((_NROWS, _PACK * DIM), jnp.float32),
    )(embT, embT, embT, embT)


def _sc_kernel(emb_hbm, i_hbm, j_hbm, out_hbm,
               idx_i_v, idx_j_v, qi_v, qj_v,
               a0_v, b0_v, a1_v, b1_v, out_v, sem0, sem1):
    wid = lax.axis_index("s") * _NC + lax.axis_index("c")
    row0 = wid * _NCHUNK

    pltpu.sync_copy(i_hbm.at[pl.ds(row0, _NCHUNK)], idx_i_v)
    pltpu.sync_copy(j_hbm.at[pl.ds(row0, _NCHUNK)], idx_j_v)

    for k in range(_NCHUNK):
        for c in range(_GRP):
            s = pl.ds(c * _LANES, _LANES)
            vi = idx_i_v[k, s]
            qi_v[k, s] = ((vi & (_Q - 1)) << 2) | (vi >> 18)
            vj = idx_j_v[k, s]
            qj_v[k, s] = ((vj & (_Q - 1)) << 2) | (vj >> 18)

    bufs = ((a0_v, b0_v, sem0), (a1_v, b1_v, sem1))

    def fire(k):
        a_v, b_v, sem = bufs[k % 2]
        return (pltpu.async_copy(emb_hbm.at[qi_v.at[k]], a_v, sem),
                pltpu.async_copy(emb_hbm.at[qj_v.at[k]], b_v, sem))

    pending = fire(0)
    for k in range(_NCHUNK):
        nxt = fire(k + 1) if k + 1 < _NCHUNK else None
        for h in pending:
            h.wait()
        a_v, b_v, _ = bufs[k % 2]

        def body(g, _, k=k, a_v=a_v, b_v=b_v):
            s = pl.ds(g * _LANES, _LANES)
            rows = g * _LANES + lax.iota(jnp.int32, _LANES)
            acc = jnp.zeros((_LANES,), jnp.float32)
            for d in range(DIM):
                col = jnp.full((_LANES,), d, jnp.int32)
                a = plsc.load_gather(a_v, [rows, col])
                b = plsc.load_gather(b_v, [rows, col])
                acc = acc + a * b
            out_v[k, s] = acc
            return 0

        lax.fori_loop(0, _GRP, body, 0)
        pending = nxt

    pltpu.sync_copy(out_v, out_hbm.at[pl.ds(row0, _NCHUNK)])


@jax.jit
def kernel(i, j, node_emb):
    embT = node_emb.T
    packed = _tc_relayout(embT)
    emb4 = packed.reshape(_PACK * _Q, DIM)
    i2 = i.astype(jnp.int32).reshape(_NW * _NCHUNK, _CHUNK)
    j2 = j.astype(jnp.int32).reshape(_NW * _NCHUNK, _CHUNK)
    mesh = plsc.VectorSubcoreMesh(core_axis_name="c", subcore_axis_name="s")
    fn = functools.partial(
        pl.kernel,
        mesh=mesh,
        out_type=jax.ShapeDtypeStruct((_NW * _NCHUNK, _CHUNK), jnp.float32),
        scratch_types=[
            pltpu.VMEM((_NCHUNK, _CHUNK), jnp.int32),
            pltpu.VMEM((_NCHUNK, _CHUNK), jnp.int32),
            pltpu.VMEM((_NCHUNK, _CHUNK), jnp.int32),
            pltpu.VMEM((_NCHUNK, _CHUNK), jnp.int32),
            pltpu.VMEM((_CHUNK, DIM), jnp.float32),
            pltpu.VMEM((_CHUNK, DIM), jnp.float32),
            pltpu.VMEM((_CHUNK, DIM), jnp.float32),
            pltpu.VMEM((_CHUNK, DIM), jnp.float32),
            pltpu.VMEM((_NCHUNK, _CHUNK), jnp.float32),
            pltpu.SemaphoreType.DMA,
            pltpu.SemaphoreType.DMA,
        ],
        compiler_params=pltpu.CompilerParams(
            use_tc_tiling_on_sc=False, needs_layout_passes=False),
    )(_sc_kernel)
    out2 = fn(emb4, i2, j2)
    return out2.reshape(BATCH)
